# baseline (speedup 1.0000x reference)
import functools

import jax
import jax.numpy as jnp
from jax import lax
from jax.experimental import pallas as pl
from jax.experimental.pallas import tpu as pltpu
from jax.experimental.pallas import tpu_sc as plsc

NUM_EXPERTS = 64
NUM_TOKENS = 32768
K = 512
TOK_BLK = 4096
LANES = 16
HSTRIDE = 256
NVREG_FULL = NUM_TOKENS // LANES


def _tc_body(nt_ref, x_ref, scores_ref, mask_ref):
    xt = x_ref[...].T
    m = jnp.max(xt, axis=0, keepdims=True)
    e = jnp.exp(xt - m)
    s = jnp.sum(e, axis=0, keepdims=True)
    scores_ref[...] = e / s

    @pl.when(pl.program_id(0) == 0)
    def _():
        col = lax.broadcasted_iota(jnp.int32, (NUM_EXPERTS, K), 1)
        mask_ref[...] = jnp.where(col < nt_ref[0], 1.0, 0.0).astype(jnp.float32)


def _tc_softmax(nt, logits):
    return pl.pallas_call(
        _tc_body,
        grid=(NUM_TOKENS // TOK_BLK,),
        in_specs=[
            pl.BlockSpec(memory_space=pltpu.SMEM),
            pl.BlockSpec((TOK_BLK, NUM_EXPERTS), lambda i: (i, 0)),
        ],
        out_specs=[
            pl.BlockSpec((NUM_EXPERTS, TOK_BLK), lambda i: (0, i)),
            pl.BlockSpec((NUM_EXPERTS, K), lambda i: (0, 0)),
        ],
        out_shape=[
            jax.ShapeDtypeStruct((NUM_EXPERTS, NUM_TOKENS), jnp.float32),
            jax.ShapeDtypeStruct((NUM_EXPERTS, K), jnp.float32),
        ],
    )(nt, logits)


def _sc_body(scores_hbm, idx_hbm, wts_hbm,
             keys_v, hist_v, tot_v, cand_k, cand_i,
             sel_k, sel_i, srt_k, srt_i, wts_v, sem_v):
    wid = lax.axis_index("s") * 2 + lax.axis_index("c")
    iota16 = jnp.arange(LANES, dtype=jnp.int32)
    ones16 = jnp.ones((LANES,), jnp.int32)
    zeros16 = jnp.zeros((LANES,), jnp.int32)

    @plsc.parallel_loop(0, HSTRIDE * LANES // LANES, unroll=4)
    def _(j):
        hist_v[pl.ds(j * LANES, LANES)] = zeros16

    def totals(nbins):
        @plsc.parallel_loop(0, nbins // LANES, unroll=2)
        def _(c):
            acc = zeros16
            for l in range(LANES):
                off = l * HSTRIDE
                acc = acc + hist_v[pl.ds(off + c * LANES, LANES)]
                hist_v[pl.ds(off + c * LANES, LANES)] = zeros16
            tot_v[pl.ds(c * LANES, LANES)] = acc

    def find_bin(nbins, r):
        nch = nbins // LANES
        carry0 = (jnp.int32(0), r, jnp.int32(0), jnp.bool_(False))

        @plsc.parallel_loop(0, nch, carry=carry0)
        def result(i, carry):
            acc, r_cur, b_found, done = carry
            c = nch - 1 - i
            tv = tot_v[pl.ds(c * LANES, LANES)]
            rv = lax.rev(tv, (0,))
            cs = plsc.cumsum(rv)
            chunk_tot = jnp.max(cs)
            cross = (acc + cs) >= r_cur
            ffs = plsc.all_reduce_ffs(cross)
            ffs_s = jnp.max(ffs)
            eq = iota16 == ffs
            cs_at = jnp.max(jnp.where(eq, cs, 0))
            tot_at = jnp.max(jnp.where(eq, rv, 0))
            has = (acc + chunk_tot) >= r_cur
            newly = jnp.logical_and(has, jnp.logical_not(done))
            b_here = c * LANES + 15 - ffs_s
            r_here = r_cur - (acc + cs_at - tot_at)
            b_found = jnp.where(newly, b_here, b_found)
            r_cur = jnp.where(newly, r_here, r_cur)
            done = jnp.logical_or(done, has)
            return acc + chunk_tot, r_cur, b_found, done

        _, r_new, b, _ = result
        return b, r_new

    def expert_body(eo, _carry):
        e = wid * 2 + eo

        pltpu.sync_copy(scores_hbm.at[e], keys_v)

        @plsc.parallel_loop(0, NVREG_FULL, unroll=8)
        def _(j):
            kv = plsc.bitcast(keys_v[pl.ds(j * LANES, LANES)], jnp.int32)
            d = kv >> 22
            plsc.addupdate_scatter(hist_v, [iota16 * HSTRIDE + d], ones16)

        totals(256)
        b1, r1 = find_bin(256, jnp.int32(K))

        thresh = b1 << 22

        @plsc.parallel_loop(0, NVREG_FULL, unroll=4,
                            carry=jnp.zeros((LANES,), jnp.int32))
        def off_vec(j, off):
            kraw = keys_v[pl.ds(j * LANES, LANES)]
            kv = plsc.bitcast(kraw, jnp.int32)
            m = kv >= thresh
            dest = off + plsc.cumsum(ones16, mask=m) - 1
            plsc.store_scatter(cand_k, [dest], kraw, mask=m)
            plsc.store_scatter(cand_i, [dest], j * LANES + iota16, mask=m)
            return off + plsc.all_reduce_population_count(m)

        cand_count = jnp.max(off_vec)
        nv_cand = (cand_count + LANES - 1) // LANES

        def cand_pass(shift_hi, prefix, shift, width, r):
            maskw = (1 << width) - 1

            @plsc.parallel_loop(0, nv_cand, unroll=4)
            def _(j):
                kv = plsc.bitcast(cand_k[pl.ds(j * LANES, LANES)], jnp.int32)
                lane_ok = (j * LANES + iota16) < cand_count
                act = jnp.logical_and((kv >> shift_hi) == prefix, lane_ok)
                d = (kv >> shift) & maskw
                plsc.addupdate_scatter(hist_v, [iota16 * HSTRIDE + d],
                                       ones16, mask=act)

            totals(2 ** width)
            b, r_new = find_bin(2 ** width, r)
            return (prefix << width) | b, r_new

        p2, r2 = cand_pass(22, b1, 14, 8, r1)
        p3, r3 = cand_pass(14, p2, 6, 8, r2)
        t_key, r4 = cand_pass(6, p3, 0, 6, r3)

        sel0 = (jnp.zeros((LANES,), jnp.int32), jnp.zeros((LANES,), jnp.int32),
                jnp.zeros((LANES,), jnp.int32))

        @plsc.parallel_loop(0, nv_cand, unroll=4, carry=sel0)
        def _sel(j, carry):
            off, ties, n1 = carry
            kraw = cand_k[pl.ds(j * LANES, LANES)]
            kv = plsc.bitcast(kraw, jnp.int32)
            iv = cand_i[pl.ds(j * LANES, LANES)]
            lane_ok = (j * LANES + iota16) < cand_count
            m_gt = jnp.logical_and(kv > t_key, lane_ok)
            m_eq = jnp.logical_and(kv == t_key, lane_ok)
            tie_rank = ties + plsc.cumsum(ones16, mask=m_eq)
            m = jnp.logical_or(m_gt, jnp.logical_and(m_eq, tie_rank <= r4))
            dest = off + plsc.cumsum(ones16, mask=m) - 1
            plsc.store_scatter(sel_k, [dest], kraw, mask=m)
            plsc.store_scatter(sel_i, [dest], iv, mask=m)
            mb = jnp.logical_and(m, (kv & 1) == 1)
            return (off + plsc.all_reduce_population_count(m),
                    ties + plsc.all_reduce_population_count(m_eq),
                    n1 + plsc.all_reduce_population_count(mb))

        n1 = _sel[2]

        bufs = [(sel_k, sel_i), (srt_k, srt_i)]
        for p in range(30):
            src_k, src_i = bufs[p % 2]
            dst_k, dst_i = bufs[(p + 1) % 2]
            carry0 = (jnp.zeros((LANES,), jnp.int32),
                      jnp.zeros((LANES,), jnp.int32))

            @plsc.parallel_loop(0, K // LANES, unroll=4, carry=carry0)
            def _st(j, c, _sk=src_k, _si=src_i, _dk=dst_k, _di=dst_i,
                    _p=p, _n1=n1):
                off1, n1n = c
                kraw = _sk[pl.ds(j * LANES, LANES)]
                kv = plsc.bitcast(kraw, jnp.int32)
                iv = _si[pl.ds(j * LANES, LANES)]
                m1 = ((kv >> _p) & 1) == 1
                m0 = jnp.logical_not(m1)
                r1 = plsc.cumsum(ones16, mask=m1)
                r0 = plsc.cumsum(ones16, mask=m0)
                d1 = off1 + r1 - 1
                d0 = _n1 + (j * LANES - off1) + r0 - 1
                dest = jnp.where(m1, d1, d0)
                plsc.store_scatter(_dk, [dest], kraw)
                plsc.store_scatter(_di, [dest], iv)
                mb = ((kv >> (_p + 1)) & 1) == 1
                return (off1 + plsc.all_reduce_population_count(m1),
                        n1n + plsc.all_reduce_population_count(mb))

            n1 = _st[1]

        @plsc.parallel_loop(0, K // LANES, carry=jnp.zeros((LANES,), jnp.float32))
        def acc(j, a):
            return a + sel_k[pl.ds(j * LANES, LANES)]

        denom = jnp.sum(acc) + jnp.float32(1e-8)

        @plsc.parallel_loop(0, K // LANES, unroll=2)
        def _(j):
            wts_v[pl.ds(j * LANES, LANES)] = sel_k[pl.ds(j * LANES, LANES)] / denom

        pltpu.sync_copy(sel_i, idx_hbm.at[e])
        pltpu.sync_copy(wts_v, wts_hbm.at[e])
        return _carry

    lax.fori_loop(0, NUM_EXPERTS // 32, expert_body, jnp.int32(0))


def _sc_topk(scores):
    mesh = plsc.VectorSubcoreMesh(core_axis_name="c", subcore_axis_name="s",
                                  num_cores=2, num_subcores=16)
    f = pl.kernel(
        _sc_body,
        out_type=(
            jax.ShapeDtypeStruct((NUM_EXPERTS, K), jnp.int32),
            jax.ShapeDtypeStruct((NUM_EXPERTS, K), jnp.float32),
        ),
        mesh=mesh,
        compiler_params=pltpu.CompilerParams(needs_layout_passes=False),
        scratch_types=[
            pltpu.VMEM((NUM_TOKENS,), jnp.float32),
            pltpu.VMEM((HSTRIDE * LANES,), jnp.int32),
            pltpu.VMEM((HSTRIDE,), jnp.int32),
            pltpu.VMEM((NUM_TOKENS + LANES,), jnp.float32),
            pltpu.VMEM((NUM_TOKENS + LANES,), jnp.int32),
            pltpu.VMEM((K,), jnp.float32),
            pltpu.VMEM((K,), jnp.int32),
            pltpu.VMEM((K,), jnp.float32),
            pltpu.VMEM((K,), jnp.int32),
            pltpu.VMEM((K,), jnp.float32),
            pltpu.SemaphoreType.DMA((8,)),
        ],
    )
    return f(scores)


def kernel(router_logits, num_tokens):
    nt = jnp.asarray(num_tokens, jnp.int32).reshape(1)
    scores, mask = _tc_softmax(nt, router_logits)
    idx, wts = _sc_topk(scores)
    return idx, wts, mask


# speedup vs baseline: 1.0424x; 1.0424x over previous
import functools

import jax
import jax.numpy as jnp
from jax import lax
from jax.experimental import pallas as pl
from jax.experimental.pallas import tpu as pltpu
from jax.experimental.pallas import tpu_sc as plsc

NUM_EXPERTS = 64
NUM_TOKENS = 32768
K = 512
TOK_BLK = 4096
LANES = 16
HSTRIDE = 256
NVREG_FULL = NUM_TOKENS // LANES


def _tc_body(nt_ref, x_ref, scores_ref, mask_ref):
    xt = x_ref[...].T
    m = jnp.max(xt, axis=0, keepdims=True)
    e = jnp.exp(xt - m)
    s = jnp.sum(e, axis=0, keepdims=True)
    scores_ref[...] = e / s

    @pl.when(pl.program_id(0) == 0)
    def _():
        col = lax.broadcasted_iota(jnp.int32, (NUM_EXPERTS, K), 1)
        mask_ref[...] = jnp.where(col < nt_ref[0], 1.0, 0.0).astype(jnp.float32)


def _tc_softmax(nt, logits):
    return pl.pallas_call(
        _tc_body,
        grid=(NUM_TOKENS // TOK_BLK,),
        in_specs=[
            pl.BlockSpec(memory_space=pltpu.SMEM),
            pl.BlockSpec((TOK_BLK, NUM_EXPERTS), lambda i: (i, 0)),
        ],
        out_specs=[
            pl.BlockSpec((NUM_EXPERTS, TOK_BLK), lambda i: (0, i)),
            pl.BlockSpec((NUM_EXPERTS, K), lambda i: (0, 0)),
        ],
        out_shape=[
            jax.ShapeDtypeStruct((NUM_EXPERTS, NUM_TOKENS), jnp.float32),
            jax.ShapeDtypeStruct((NUM_EXPERTS, K), jnp.float32),
        ],
    )(nt, logits)


def _sc_body(scores_hbm, idx_hbm, wts_hbm,
             keys_v, hist_v, tot_v, cand_k, cand_i,
             sel_k, sel_i, srt_k, srt_i, wts_v, sem_v):
    wid = lax.axis_index("s") * 2 + lax.axis_index("c")
    iota16 = jnp.arange(LANES, dtype=jnp.int32)
    ones16 = jnp.ones((LANES,), jnp.int32)
    zeros16 = jnp.zeros((LANES,), jnp.int32)

    @plsc.parallel_loop(0, HSTRIDE * LANES // LANES, unroll=4)
    def _(j):
        hist_v[pl.ds(j * LANES, LANES)] = zeros16

    def totals(nbins):
        @plsc.parallel_loop(0, nbins // LANES, unroll=2)
        def _(c):
            acc = zeros16
            for l in range(LANES):
                off = l * HSTRIDE
                acc = acc + hist_v[pl.ds(off + c * LANES, LANES)]
                hist_v[pl.ds(off + c * LANES, LANES)] = zeros16
            tot_v[pl.ds(c * LANES, LANES)] = acc

    def find_bin(nbins, r):
        nch = nbins // LANES
        carry0 = (jnp.int32(0), r, jnp.int32(0), jnp.bool_(False))

        @plsc.parallel_loop(0, nch, carry=carry0)
        def result(i, carry):
            acc, r_cur, b_found, done = carry
            c = nch - 1 - i
            tv = tot_v[pl.ds(c * LANES, LANES)]
            rv = lax.rev(tv, (0,))
            cs = plsc.cumsum(rv)
            chunk_tot = jnp.max(cs)
            cross = (acc + cs) >= r_cur
            ffs = plsc.all_reduce_ffs(cross)
            ffs_s = jnp.max(ffs)
            eq = iota16 == ffs
            cs_at = jnp.max(jnp.where(eq, cs, 0))
            tot_at = jnp.max(jnp.where(eq, rv, 0))
            has = (acc + chunk_tot) >= r_cur
            newly = jnp.logical_and(has, jnp.logical_not(done))
            b_here = c * LANES + 15 - ffs_s
            r_here = r_cur - (acc + cs_at - tot_at)
            b_found = jnp.where(newly, b_here, b_found)
            r_cur = jnp.where(newly, r_here, r_cur)
            done = jnp.logical_or(done, has)
            return acc + chunk_tot, r_cur, b_found, done

        _, r_new, b, _ = result
        return b, r_new

    def expert_body(eo, _carry):
        e = wid * 2 + eo

        pltpu.sync_copy(scores_hbm.at[e], keys_v)

        @plsc.parallel_loop(0, NVREG_FULL, unroll=8)
        def _(j):
            kv = plsc.bitcast(keys_v[pl.ds(j * LANES, LANES)], jnp.int32)
            d = kv >> 22
            plsc.addupdate_scatter(hist_v, [iota16 * HSTRIDE + d], ones16)

        totals(256)
        b1, r1 = find_bin(256, jnp.int32(K))

        thresh = b1 << 22

        @plsc.parallel_loop(0, NVREG_FULL, unroll=4,
                            carry=jnp.zeros((LANES,), jnp.int32))
        def off_vec(j, off):
            kraw = keys_v[pl.ds(j * LANES, LANES)]
            kv = plsc.bitcast(kraw, jnp.int32)
            m = kv >= thresh
            dest = off + plsc.cumsum(ones16, mask=m) - 1
            plsc.store_scatter(cand_k, [dest], kraw, mask=m)
            plsc.store_scatter(cand_i, [dest], j * LANES + iota16, mask=m)
            return off + plsc.all_reduce_population_count(m)

        cand_count = jnp.max(off_vec)
        nv_cand = (cand_count + LANES - 1) // LANES

        def cand_pass(shift_hi, prefix, shift, width, r):
            maskw = (1 << width) - 1

            @plsc.parallel_loop(0, nv_cand, unroll=4)
            def _(j):
                kv = plsc.bitcast(cand_k[pl.ds(j * LANES, LANES)], jnp.int32)
                lane_ok = (j * LANES + iota16) < cand_count
                act = jnp.logical_and((kv >> shift_hi) == prefix, lane_ok)
                d = (kv >> shift) & maskw
                plsc.addupdate_scatter(hist_v, [iota16 * HSTRIDE + d],
                                       ones16, mask=act)

            totals(2 ** width)
            b, r_new = find_bin(2 ** width, r)
            return (prefix << width) | b, r_new

        p2, r2 = cand_pass(22, b1, 14, 8, r1)
        p3, r3 = cand_pass(14, p2, 6, 8, r2)
        t_key, r4 = cand_pass(6, p3, 0, 6, r3)

        sel0 = (jnp.zeros((LANES,), jnp.int32), jnp.zeros((LANES,), jnp.int32),
                jnp.zeros((LANES,), jnp.int32))

        @plsc.parallel_loop(0, nv_cand, unroll=4, carry=sel0)
        def _sel(j, carry):
            off, ties, n1 = carry
            kraw = cand_k[pl.ds(j * LANES, LANES)]
            kv = plsc.bitcast(kraw, jnp.int32)
            iv = cand_i[pl.ds(j * LANES, LANES)]
            lane_ok = (j * LANES + iota16) < cand_count
            m_gt = jnp.logical_and(kv > t_key, lane_ok)
            m_eq = jnp.logical_and(kv == t_key, lane_ok)
            tie_rank = ties + plsc.cumsum(ones16, mask=m_eq)
            m = jnp.logical_or(m_gt, jnp.logical_and(m_eq, tie_rank <= r4))
            dest = off + plsc.cumsum(ones16, mask=m) - 1
            plsc.store_scatter(sel_k, [dest], kraw, mask=m)
            plsc.store_scatter(sel_i, [dest], iv, mask=m)
            mb = jnp.logical_and(m, (kv & 1) == 1)
            return (off + plsc.all_reduce_population_count(m),
                    ties + plsc.all_reduce_population_count(m_eq),
                    n1 + plsc.all_reduce_population_count(mb))

        n1 = _sel[2]

        bufs = [(sel_k, sel_i), (srt_k, srt_i)]
        for p in range(30):
            src_k, src_i = bufs[p % 2]
            dst_k, dst_i = bufs[(p + 1) % 2]
            carry0 = (jnp.zeros((LANES,), jnp.int32),
                      jnp.zeros((LANES,), jnp.int32))

            @plsc.parallel_loop(0, K // LANES, unroll=2, carry=carry0)
            def _st(j, c, _sk=src_k, _si=src_i, _dk=dst_k, _di=dst_i,
                    _p=p, _n1=n1):
                off1, n1n = c
                kraw = _sk[pl.ds(j * LANES, LANES)]
                kv = plsc.bitcast(kraw, jnp.int32)
                iv = _si[pl.ds(j * LANES, LANES)]
                m1 = ((kv >> _p) & 1) == 1
                m0 = jnp.logical_not(m1)
                r1 = plsc.cumsum(ones16, mask=m1)
                r0 = plsc.cumsum(ones16, mask=m0)
                d1 = off1 + r1 - 1
                d0 = _n1 + (j * LANES - off1) + r0 - 1
                dest = jnp.where(m1, d1, d0)
                plsc.store_scatter(_dk, [dest], kraw)
                plsc.store_scatter(_di, [dest], iv)
                mb = ((kv >> (_p + 1)) & 1) == 1
                return (off1 + plsc.all_reduce_population_count(m1),
                        n1n + plsc.all_reduce_population_count(mb))

            n1 = _st[1]

        @plsc.parallel_loop(0, K // LANES, carry=jnp.zeros((LANES,), jnp.float32))
        def acc(j, a):
            return a + sel_k[pl.ds(j * LANES, LANES)]

        denom = jnp.sum(acc) + jnp.float32(1e-8)

        @plsc.parallel_loop(0, K // LANES, unroll=2)
        def _(j):
            wts_v[pl.ds(j * LANES, LANES)] = sel_k[pl.ds(j * LANES, LANES)] / denom

        pltpu.sync_copy(sel_i, idx_hbm.at[e])
        pltpu.sync_copy(wts_v, wts_hbm.at[e])
        return _carry

    lax.fori_loop(0, NUM_EXPERTS // 32, expert_body, jnp.int32(0))


def _sc_topk(scores):
    mesh = plsc.VectorSubcoreMesh(core_axis_name="c", subcore_axis_name="s",
                                  num_cores=2, num_subcores=16)
    f = pl.kernel(
        _sc_body,
        out_type=(
            jax.ShapeDtypeStruct((NUM_EXPERTS, K), jnp.int32),
            jax.ShapeDtypeStruct((NUM_EXPERTS, K), jnp.float32),
        ),
        mesh=mesh,
        compiler_params=pltpu.CompilerParams(needs_layout_passes=False),
        scratch_types=[
            pltpu.VMEM((NUM_TOKENS,), jnp.float32),
            pltpu.VMEM((HSTRIDE * LANES,), jnp.int32),
            pltpu.VMEM((HSTRIDE,), jnp.int32),
            pltpu.VMEM((NUM_TOKENS + LANES,), jnp.float32),
            pltpu.VMEM((NUM_TOKENS + LANES,), jnp.int32),
            pltpu.VMEM((K,), jnp.float32),
            pltpu.VMEM((K,), jnp.int32),
            pltpu.VMEM((K,), jnp.float32),
            pltpu.VMEM((K,), jnp.int32),
            pltpu.VMEM((K,), jnp.float32),
            pltpu.SemaphoreType.DMA((8,)),
        ],
    )
    return f(scores)


def kernel(router_logits, num_tokens):
    nt = jnp.asarray(num_tokens, jnp.int32).reshape(1)
    scores, mask = _tc_softmax(nt, router_logits)
    idx, wts = _sc_topk(scores)
    return idx, wts, mask
